# Initial kernel scaffold; baseline (speedup 1.0000x reference)
import numpy as np
import jax
import jax.numpy as jnp
from jax.experimental import pallas as pl
from jax.experimental.pallas import tpu as pltpu

CB = 7
IMG = 256


def _randomize_betas_const(cb):
    rnd_lst = np.random.choice(2, cb, p=[0.5, 0.5]).tolist()
    for i in range(len(rnd_lst)):
        ln = len(rnd_lst) - 1
        if rnd_lst[i] == 0:
            rnd_lst[i] = rnd_lst[i] + i
        else:
            rnd_lst[i] = rnd_lst[i] + ln + i
    return rnd_lst


np.random.seed(0)
_SELS = [_randomize_betas_const(CB) for _ in range(4)]
_THETA_SEL = np.random.choice(2, 4, p=[0.5, 0.5]).tolist()
_BITS_PACKED = [
    sum((1 << c) for c in range(CB) if sel[c] >= CB) for sel in _SELS
]


def _recon_kernel(b1_ref, b2_ref, t1_ref, t2_ref, o1_ref, o2_ref, o3_ref, o4_ref):
    b = pl.program_id(0)
    c = pl.program_id(1)
    outs = (o1_ref, o2_ref, o3_ref, o4_ref)

    @pl.when(c < CB)
    def _():
        v1 = b1_ref[0, 0]
        v2 = b2_ref[0, 0]
        for k in range(4):
            bit = jax.lax.rem(jax.lax.shift_right_logical(_BITS_PACKED[k], c), 2)
            outs[k][0, 0] = jnp.where(bit == 1, v2, v1)

    @pl.when(c == CB)
    def _():
        for k in range(4):
            t_ref = t2_ref if _THETA_SEL[k] else t1_ref
            outs[k][0, 0] = jnp.full((IMG, IMG), t_ref[b, 0], jnp.float32)


def kernel(beta1, beta2, theta1, theta2):
    batch = beta1.shape[0]
    beta_spec = pl.BlockSpec(
        (1, 1, IMG, IMG),
        lambda b, c: (b, jnp.minimum(c, CB - 1), 0, 0),
    )
    theta_spec = pl.BlockSpec(memory_space=pltpu.SMEM)
    out_spec = pl.BlockSpec((1, 1, IMG, IMG), lambda b, c: (b, c, 0, 0))
    out_shape = jax.ShapeDtypeStruct((batch, CB + 1, IMG, IMG), jnp.float32)

    outs = pl.pallas_call(
        _recon_kernel,
        grid=(batch, CB + 1),
        in_specs=[beta_spec, beta_spec, theta_spec, theta_spec],
        out_specs=[out_spec] * 4,
        out_shape=[out_shape] * 4,
    )(beta1, beta2, theta1, theta2)

    return (*outs, jnp.array(_THETA_SEL, dtype=jnp.int32))


# speedup vs baseline: 3.2517x; 3.2517x over previous
import numpy as np
import jax
import jax.numpy as jnp
from jax.experimental import pallas as pl
from jax.experimental.pallas import tpu as pltpu

CB = 7
IMG = 256


def _randomize_betas_const(cb):
    rnd_lst = np.random.choice(2, cb, p=[0.5, 0.5]).tolist()
    for i in range(len(rnd_lst)):
        ln = len(rnd_lst) - 1
        if rnd_lst[i] == 0:
            rnd_lst[i] = rnd_lst[i] + i
        else:
            rnd_lst[i] = rnd_lst[i] + ln + i
    return rnd_lst


np.random.seed(0)
_SELS = [_randomize_betas_const(CB) for _ in range(4)]
_THETA_SEL = np.random.choice(2, 4, p=[0.5, 0.5]).tolist()
_BITS_PACKED = [
    sum((1 << c) for c in range(CB) if sel[c] >= CB) for sel in _SELS
]


def _recon_kernel(b1_ref, b2_ref, t1_ref, t2_ref, o1_ref, o2_ref, o3_ref, o4_ref):
    c = pl.program_id(1)
    outs = (o1_ref, o2_ref, o3_ref, o4_ref)

    @pl.when(c < CB)
    def _():
        v1 = b1_ref[0, 0]
        v2 = b2_ref[0, 0]
        for k in range(4):
            bit = jax.lax.rem(jax.lax.shift_right_logical(_BITS_PACKED[k], c), 2)
            outs[k][0, 0] = jnp.where(bit == 1, v2, v1)

    @pl.when(c == CB)
    def _():
        for k in range(4):
            t_ref = t2_ref if _THETA_SEL[k] else t1_ref
            outs[k][0, 0] = jnp.broadcast_to(t_ref[:, :], (IMG, IMG))


def kernel(beta1, beta2, theta1, theta2):
    batch = beta1.shape[0]
    t1_col = jnp.tile(theta1[:, 0], IMG // batch).reshape(IMG, 1)
    t2_col = jnp.tile(theta2[:, 0], IMG // batch).reshape(IMG, 1)

    beta_spec = pl.BlockSpec(
        (1, 1, IMG, IMG),
        lambda b, c: (b, jnp.minimum(c, CB - 1), 0, 0),
    )
    theta_spec = pl.BlockSpec((IMG, 1), lambda b, c: (0, 0))
    out_spec = pl.BlockSpec((1, 1, IMG, IMG), lambda b, c: (b, c, 0, 0))
    out_shape = jax.ShapeDtypeStruct((batch, CB + 1, IMG, IMG), jnp.float32)

    outs = pl.pallas_call(
        _recon_kernel,
        grid=(batch, CB + 1),
        in_specs=[beta_spec, beta_spec, theta_spec, theta_spec],
        out_specs=[out_spec] * 4,
        out_shape=[out_shape] * 4,
    )(beta1, beta2, t1_col, t2_col)

    return (*outs, jnp.array(_THETA_SEL, dtype=jnp.int32))


# speedup vs baseline: 5.6004x; 1.7223x over previous
import numpy as np
import jax
import jax.numpy as jnp
from jax.experimental import pallas as pl
from jax.experimental.pallas import tpu as pltpu

CB = 7
IMG = 256


def _randomize_betas_const(cb):
    rnd_lst = np.random.choice(2, cb, p=[0.5, 0.5]).tolist()
    for i in range(len(rnd_lst)):
        ln = len(rnd_lst) - 1
        if rnd_lst[i] == 0:
            rnd_lst[i] = rnd_lst[i] + i
        else:
            rnd_lst[i] = rnd_lst[i] + ln + i
    return rnd_lst


np.random.seed(0)
_SELS = [_randomize_betas_const(CB) for _ in range(4)]
_THETA_SEL = np.random.choice(2, 4, p=[0.5, 0.5]).tolist()
_BITS_PACKED = [
    sum((1 << c) for c in range(CB) if sel[c] >= CB) for sel in _SELS
]


BB = 4


def _recon_kernel(b1_ref, b2_ref, t1_ref, t2_ref, o1_ref, o2_ref, o3_ref, o4_ref):
    c = pl.program_id(1)
    outs = (o1_ref, o2_ref, o3_ref, o4_ref)

    @pl.when(c < CB)
    def _():
        v1 = b1_ref[:, 0]
        v2 = b2_ref[:, 0]
        for k in range(4):
            bit = jax.lax.rem(jax.lax.shift_right_logical(_BITS_PACKED[k], c), 2)
            outs[k][:, 0] = jnp.where(bit == 1, v2, v1)

    @pl.when(c == CB)
    def _():
        for k in range(4):
            t_ref = t2_ref if _THETA_SEL[k] else t1_ref
            tcol = t_ref[:, :]
            outs[k][:, 0] = jnp.broadcast_to(tcol[None, :, :], (BB, IMG, IMG))


def kernel(beta1, beta2, theta1, theta2):
    batch = beta1.shape[0]
    t1_col = jnp.tile(theta1[:, 0], IMG // batch).reshape(IMG, 1)
    t2_col = jnp.tile(theta2[:, 0], IMG // batch).reshape(IMG, 1)

    beta_spec = pl.BlockSpec(
        (BB, 1, IMG, IMG),
        lambda b, c: (b, jnp.minimum(c, CB - 1), 0, 0),
    )
    theta_spec = pl.BlockSpec((IMG, 1), lambda b, c: (0, 0))
    out_spec = pl.BlockSpec((BB, 1, IMG, IMG), lambda b, c: (b, c, 0, 0))
    out_shape = jax.ShapeDtypeStruct((batch, CB + 1, IMG, IMG), jnp.float32)

    outs = pl.pallas_call(
        _recon_kernel,
        grid=(batch // BB, CB + 1),
        in_specs=[beta_spec, beta_spec, theta_spec, theta_spec],
        out_specs=[out_spec] * 4,
        out_shape=[out_shape] * 4,
    )(beta1, beta2, t1_col, t2_col)

    return (*outs, jnp.array(_THETA_SEL, dtype=jnp.int32))


# speedup vs baseline: 6.0615x; 1.0823x over previous
import numpy as np
import jax
import jax.numpy as jnp
from jax.experimental import pallas as pl
from jax.experimental.pallas import tpu as pltpu

CB = 7
IMG = 256


def _randomize_betas_const(cb):
    rnd_lst = np.random.choice(2, cb, p=[0.5, 0.5]).tolist()
    for i in range(len(rnd_lst)):
        ln = len(rnd_lst) - 1
        if rnd_lst[i] == 0:
            rnd_lst[i] = rnd_lst[i] + i
        else:
            rnd_lst[i] = rnd_lst[i] + ln + i
    return rnd_lst


np.random.seed(0)
_SELS = [_randomize_betas_const(CB) for _ in range(4)]
_THETA_SEL = np.random.choice(2, 4, p=[0.5, 0.5]).tolist()
_BITS_PACKED = [
    sum((1 << c) for c in range(CB) if sel[c] >= CB) for sel in _SELS
]


BB = 8


def _recon_kernel(b1_ref, b2_ref, t1_ref, t2_ref, o1_ref, o2_ref, o3_ref, o4_ref):
    c = pl.program_id(1)
    outs = (o1_ref, o2_ref, o3_ref, o4_ref)

    @pl.when(c < CB)
    def _():
        v1 = b1_ref[:, 0]
        v2 = b2_ref[:, 0]
        for k in range(4):
            bit = jax.lax.rem(jax.lax.shift_right_logical(_BITS_PACKED[k], c), 2)
            outs[k][:, 0] = jnp.where(bit == 1, v2, v1)

    @pl.when(c == CB)
    def _():
        for k in range(4):
            t_ref = t2_ref if _THETA_SEL[k] else t1_ref
            tcol = t_ref[:, :]
            outs[k][:, 0] = jnp.broadcast_to(tcol[None, :, :], (BB, IMG, IMG))


def kernel(beta1, beta2, theta1, theta2):
    batch = beta1.shape[0]
    t1_col = jnp.tile(theta1[:, 0], IMG // batch).reshape(IMG, 1)
    t2_col = jnp.tile(theta2[:, 0], IMG // batch).reshape(IMG, 1)

    beta_spec = pl.BlockSpec(
        (BB, 1, IMG, IMG),
        lambda b, c: (b, jnp.minimum(c, CB - 1), 0, 0),
    )
    theta_spec = pl.BlockSpec((IMG, 1), lambda b, c: (0, 0))
    out_spec = pl.BlockSpec((BB, 1, IMG, IMG), lambda b, c: (b, c, 0, 0))
    out_shape = jax.ShapeDtypeStruct((batch, CB + 1, IMG, IMG), jnp.float32)

    outs = pl.pallas_call(
        _recon_kernel,
        grid=(batch // BB, CB + 1),
        in_specs=[beta_spec, beta_spec, theta_spec, theta_spec],
        out_specs=[out_spec] * 4,
        out_shape=[out_shape] * 4,
    )(beta1, beta2, t1_col, t2_col)

    return (*outs, jnp.array(_THETA_SEL, dtype=jnp.int32))


# speedup vs baseline: 6.2786x; 1.0358x over previous
import numpy as np
import jax
import jax.numpy as jnp
from jax.experimental import pallas as pl
from jax.experimental.pallas import tpu as pltpu

CB = 7
IMG = 256


def _randomize_betas_const(cb):
    rnd_lst = np.random.choice(2, cb, p=[0.5, 0.5]).tolist()
    for i in range(len(rnd_lst)):
        ln = len(rnd_lst) - 1
        if rnd_lst[i] == 0:
            rnd_lst[i] = rnd_lst[i] + i
        else:
            rnd_lst[i] = rnd_lst[i] + ln + i
    return rnd_lst


np.random.seed(0)
_SELS = [_randomize_betas_const(CB) for _ in range(4)]
_THETA_SEL = np.random.choice(2, 4, p=[0.5, 0.5]).tolist()
_BITS_PACKED = [
    sum((1 << c) for c in range(CB) if sel[c] >= CB) for sel in _SELS
]


BB = 16


def _recon_kernel(b1_ref, b2_ref, t1_ref, t2_ref, o1_ref, o2_ref, o3_ref, o4_ref):
    c = pl.program_id(1)
    outs = (o1_ref, o2_ref, o3_ref, o4_ref)

    @pl.when(c < CB)
    def _():
        v1 = b1_ref[:, 0]
        v2 = b2_ref[:, 0]
        for k in range(4):
            bit = jax.lax.rem(jax.lax.shift_right_logical(_BITS_PACKED[k], c), 2)
            outs[k][:, 0] = jnp.where(bit == 1, v2, v1)

    @pl.when(c == CB)
    def _():
        for k in range(4):
            t_ref = t2_ref if _THETA_SEL[k] else t1_ref
            tcol = t_ref[:, :]
            outs[k][:, 0] = jnp.broadcast_to(tcol[None, :, :], (BB, IMG, IMG))


def kernel(beta1, beta2, theta1, theta2):
    batch = beta1.shape[0]
    t1_col = jnp.tile(theta1[:, 0], IMG // batch).reshape(IMG, 1)
    t2_col = jnp.tile(theta2[:, 0], IMG // batch).reshape(IMG, 1)

    beta_spec = pl.BlockSpec(
        (BB, 1, IMG, IMG),
        lambda b, c: (b, jnp.minimum(c, CB - 1), 0, 0),
    )
    theta_spec = pl.BlockSpec((IMG, 1), lambda b, c: (0, 0))
    out_spec = pl.BlockSpec((BB, 1, IMG, IMG), lambda b, c: (b, c, 0, 0))
    out_shape = jax.ShapeDtypeStruct((batch, CB + 1, IMG, IMG), jnp.float32)

    outs = pl.pallas_call(
        _recon_kernel,
        grid=(batch // BB, CB + 1),
        in_specs=[beta_spec, beta_spec, theta_spec, theta_spec],
        out_specs=[out_spec] * 4,
        out_shape=[out_shape] * 4,
    )(beta1, beta2, t1_col, t2_col)

    return (*outs, jnp.array(_THETA_SEL, dtype=jnp.int32))
